# Initial kernel scaffold; baseline (speedup 1.0000x reference)
import jax
import jax.numpy as jnp
from jax.experimental import pallas as pl


def kernel(x, idxs, warp):
    raise NotImplementedError("write your pallas kernel here")


# speedup vs baseline: 6.6267x; 6.6267x over previous
import jax
import jax.numpy as jnp
from jax.experimental import pallas as pl
from jax.experimental.pallas import tpu as pltpu

CHANS, SEQ, FEAT = 8, 4096, 1024
N_ROWS = SEQ // 4
BLOCK_S = 512
SEQ_BLOCKS = SEQ // BLOCK_S


def _row_scale_body(idx_ref, warp_ref, x_ref, out_ref, scale_ref):
    c = pl.program_id(1)

    @pl.when(c == 0)
    def _compute_scale():
        s = pl.program_id(0)
        rows = jax.lax.broadcasted_iota(jnp.int32, (BLOCK_S, 1), 0) + s * BLOCK_S
        eq = rows == idx_ref[...]
        contrib = jnp.where(eq, warp_ref[...] - 1.0, 0.0)
        scale_ref[...] = 1.0 + jnp.sum(contrib, axis=1, keepdims=True)

    out_ref[...] = x_ref[...] * scale_ref[...][None, :, :]


def kernel(x, idxs, warp):
    idxs2d = idxs.reshape(1, N_ROWS)
    warp2d = warp.reshape(1, N_ROWS)
    return pl.pallas_call(
        _row_scale_body,
        grid=(SEQ_BLOCKS, CHANS),
        in_specs=[
            pl.BlockSpec((1, N_ROWS), lambda s, c: (0, 0)),
            pl.BlockSpec((1, N_ROWS), lambda s, c: (0, 0)),
            pl.BlockSpec((1, BLOCK_S, FEAT), lambda s, c: (c, s, 0)),
        ],
        out_specs=pl.BlockSpec((1, BLOCK_S, FEAT), lambda s, c: (c, s, 0)),
        out_shape=jax.ShapeDtypeStruct((CHANS, SEQ, FEAT), x.dtype),
        scratch_shapes=[pltpu.VMEM((BLOCK_S, 1), jnp.float32)],
        compiler_params=pltpu.CompilerParams(
            dimension_semantics=("arbitrary", "arbitrary"),
        ),
    )(idxs2d, warp2d, x)
